# baseline (device time: 37798 ns/iter reference)
import functools

import jax
import jax.numpy as jnp
from jax import lax
from jax.experimental import pallas as pl
from jax.experimental.pallas import tpu as pltpu

N_DEV = 16


def kernel(x, w_mat):
    m, _ = x.shape
    _, n = w_mat.shape
    rows = m // N_DEV

    def body(x_ref, w_ref, out_ref, p_ref, recv_ref, send_sems, recv_sems):
        my = lax.axis_index("i")

        p = jnp.dot(
            x_ref[...].astype(jnp.bfloat16),
            w_ref[...].astype(jnp.bfloat16),
            preferred_element_type=jnp.float32,
        )
        p_ref[...] = p.astype(jnp.bfloat16).reshape(N_DEV, rows, n)

        barrier_sem = pltpu.get_barrier_semaphore()
        for s in range(1, N_DEV):
            pl.semaphore_signal(
                barrier_sem,
                inc=1,
                device_id=((my + s) % N_DEV,),
                device_id_type=pl.DeviceIdType.MESH,
            )
        pl.semaphore_wait(barrier_sem, N_DEV - 1)

        rdmas = []
        for s in range(1, N_DEV):
            t = (my + s) % N_DEV
            rdma = pltpu.make_async_remote_copy(
                src_ref=p_ref.at[t],
                dst_ref=recv_ref.at[s - 1],
                send_sem=send_sems.at[s - 1],
                recv_sem=recv_sems.at[s - 1],
                device_id=(t,),
                device_id_type=pl.DeviceIdType.MESH,
            )
            rdma.start()
            rdmas.append(rdma)

        acc = p_ref[my].astype(jnp.float32)
        for k in range(N_DEV - 1):
            rdmas[k].wait_recv()
            acc = acc + recv_ref[k].astype(jnp.float32)

        out_ref[...] = acc / (1.0 + jnp.exp(-acc))

        for rdma in rdmas:
            rdma.wait_send()

        @functools.partial(pl.run_scoped, exit_sem=pltpu.SemaphoreType.REGULAR)
        def _(exit_sem):
            for s in range(1, N_DEV):
                pl.semaphore_signal(
                    exit_sem,
                    inc=1,
                    device_id=((my + s) % N_DEV,),
                    device_id_type=pl.DeviceIdType.MESH,
                )
            pl.semaphore_wait(exit_sem, N_DEV - 1)

    return pl.pallas_call(
        body,
        out_shape=jax.ShapeDtypeStruct((rows, n), jnp.float32),
        in_specs=[
            pl.BlockSpec(memory_space=pltpu.VMEM),
            pl.BlockSpec(memory_space=pltpu.VMEM),
        ],
        out_specs=pl.BlockSpec(memory_space=pltpu.VMEM),
        scratch_shapes=[
            pltpu.VMEM((N_DEV, rows, n), jnp.bfloat16),
            pltpu.VMEM((N_DEV - 1, rows, n), jnp.bfloat16),
            pltpu.SemaphoreType.DMA((N_DEV - 1,)),
            pltpu.SemaphoreType.DMA((N_DEV - 1,)),
        ],
        compiler_params=pltpu.CompilerParams(collective_id=0),
    )(x, w_mat)


# device time: 37656 ns/iter; 1.0038x vs baseline; 1.0038x over previous
import functools

import jax
import jax.numpy as jnp
from jax import lax
from jax.experimental import pallas as pl
from jax.experimental.pallas import tpu as pltpu

N_DEV = 16

_QX = (1, 0, 3, 2)
_QY = (3, 2, 1, 0)
_SELFQ = ((0, 3), (1, 2), (1, 2), (0, 3))
_OTHQ = ((1, 2), (0, 3), (0, 3), (1, 2))


def kernel(x, w_mat):
    m, _ = x.shape
    _, n = w_mat.shape
    rows = m // N_DEV

    def body(
        x_ref,
        w_ref,
        out_ref,
        p_ref,
        rx_ref,
        ry_ref,
        rz_ref,
        sx_send,
        sx_recv,
        sy_send,
        sy_recv,
        sz_send,
        sz_recv,
    ):
        my = lax.axis_index("i")
        myq = my % 4
        myz = my // 4
        qx_t = myq ^ 1
        qy_t = 3 - myq
        px = 4 * myz + qx_t
        py = 4 * myz + qy_t
        s0 = jnp.minimum(myq, 3 - myq)
        selfq = [s0, 3 - s0]
        o0 = jnp.minimum(qx_t, 3 - qx_t)
        othq = [o0, 3 - o0]

        p = jnp.dot(
            x_ref[...].astype(jnp.bfloat16),
            w_ref[...].astype(jnp.bfloat16),
            preferred_element_type=jnp.float32,
        )
        p_ref[...] = p.astype(jnp.bfloat16).reshape(N_DEV, rows, n)

        zmates = [4 * jnp.where(j >= myz, j + 1, j) + myq for j in range(3)]
        peers = [px, py] + zmates
        barrier_sem = pltpu.get_barrier_semaphore()
        for t in peers:
            pl.semaphore_signal(
                barrier_sem,
                inc=1,
                device_id=(t,),
                device_id_type=pl.DeviceIdType.MESH,
            )
        pl.semaphore_wait(barrier_sem, len(peers))

        rdmas_x = []
        for j in range(8):
            c = 4 * (j // 2) + othq[j % 2]
            rdma = pltpu.make_async_remote_copy(
                src_ref=p_ref.at[c],
                dst_ref=rx_ref.at[j],
                send_sem=sx_send.at[j],
                recv_sem=sx_recv.at[j],
                device_id=(px,),
                device_id_type=pl.DeviceIdType.MESH,
            )
            rdma.start()
            rdmas_x.append(rdma)
        for j in range(8):
            rdmas_x[j].wait_recv()
            c = 4 * (j // 2) + selfq[j % 2]
            p_ref[c] = p_ref[c] + rx_ref[j]

        rdmas_y = []
        for z in range(4):
            c = 4 * z + qy_t
            rdma = pltpu.make_async_remote_copy(
                src_ref=p_ref.at[c],
                dst_ref=ry_ref.at[z],
                send_sem=sy_send.at[z],
                recv_sem=sy_recv.at[z],
                device_id=(py,),
                device_id_type=pl.DeviceIdType.MESH,
            )
            rdma.start()
            rdmas_y.append(rdma)
        for z in range(4):
            rdmas_y[z].wait_recv()
            c = 4 * z + myq
            p_ref[c] = p_ref[c] + ry_ref[z]

        rdmas_z = []
        for j in range(3):
            zp = jnp.where(j >= myz, j + 1, j)
            dest = 4 * zp + myq
            slot = jnp.where(myz > zp, myz - 1, myz)
            rdma = pltpu.make_async_remote_copy(
                src_ref=p_ref.at[dest],
                dst_ref=rz_ref.at[slot],
                send_sem=sz_send.at[j],
                recv_sem=sz_recv.at[slot],
                device_id=(dest,),
                device_id_type=pl.DeviceIdType.MESH,
            )
            rdma.start()
            rdmas_z.append(rdma)

        acc = p_ref[my].astype(jnp.float32)
        for k in range(3):
            recv = pltpu.make_async_remote_copy(
                src_ref=p_ref.at[0],
                dst_ref=rz_ref.at[k],
                send_sem=sz_send.at[0],
                recv_sem=sz_recv.at[k],
                device_id=(my,),
                device_id_type=pl.DeviceIdType.MESH,
            )
            recv.wait_recv()
            acc = acc + rz_ref[k].astype(jnp.float32)

        out_ref[...] = acc / (1.0 + jnp.exp(-acc))

        for rdma in rdmas_x + rdmas_y + rdmas_z:
            rdma.wait_send()

        @functools.partial(pl.run_scoped, exit_sem=pltpu.SemaphoreType.REGULAR)
        def _(exit_sem):
            for t in peers:
                pl.semaphore_signal(
                    exit_sem,
                    inc=1,
                    device_id=(t,),
                    device_id_type=pl.DeviceIdType.MESH,
                )
            pl.semaphore_wait(exit_sem, len(peers))

    return pl.pallas_call(
        body,
        out_shape=jax.ShapeDtypeStruct((rows, n), jnp.float32),
        in_specs=[
            pl.BlockSpec(memory_space=pltpu.VMEM),
            pl.BlockSpec(memory_space=pltpu.VMEM),
        ],
        out_specs=pl.BlockSpec(memory_space=pltpu.VMEM),
        scratch_shapes=[
            pltpu.VMEM((N_DEV, rows, n), jnp.bfloat16),
            pltpu.VMEM((8, rows, n), jnp.bfloat16),
            pltpu.VMEM((4, rows, n), jnp.bfloat16),
            pltpu.VMEM((3, rows, n), jnp.bfloat16),
            pltpu.SemaphoreType.DMA((8,)),
            pltpu.SemaphoreType.DMA((8,)),
            pltpu.SemaphoreType.DMA((4,)),
            pltpu.SemaphoreType.DMA((4,)),
            pltpu.SemaphoreType.DMA((3,)),
            pltpu.SemaphoreType.DMA((3,)),
        ],
        compiler_params=pltpu.CompilerParams(collective_id=0),
    )(x, w_mat)


# device time: 37602 ns/iter; 1.0052x vs baseline; 1.0014x over previous
import functools

import jax
import jax.numpy as jnp
from jax import lax
from jax.experimental import pallas as pl
from jax.experimental.pallas import tpu as pltpu

N_DEV = 16

_QXY = ((0, 0), (1, 0), (1, 1), (0, 1))
_XY2Q = {(x, y): q for q, (x, y) in enumerate(_QXY)}


def kernel(x, w_mat):
    m, _ = x.shape
    _, n = w_mat.shape
    rows = m // N_DEV

    def body(
        x_ref,
        w_ref,
        out_ref,
        p_ref,
        rx_ref,
        ry_ref,
        rz_ref,
        sx_send,
        sx_recv,
        sy_send,
        sy_recv,
        sz_send,
        sz_recv,
    ):
        my = lax.axis_index("i")
        myq = my % 4
        myz = my // 4
        myx = ((myq == 1) | (myq == 2)).astype(jnp.int32)
        myy = (myq >= 2).astype(jnp.int32)
        px = 4 * myz + (myq ^ 1)
        py = 4 * myz + (3 - myq)

        p = jnp.dot(
            x_ref[...].astype(jnp.bfloat16),
            w_ref[...].astype(jnp.bfloat16),
            preferred_element_type=jnp.float32,
        ).astype(jnp.bfloat16)
        for ell in range(N_DEV):
            xg, yg, z = ell // 8, (ell % 8) // 4, ell % 4
            d = 4 * z + _XY2Q[(xg, yg)]
            p_ref[ell] = p[d * rows : (d + 1) * rows, :]

        zmates = [4 * jnp.where(j >= myz, j + 1, j) + myq for j in range(3)]
        peers = [px, py] + zmates
        barrier_sem = pltpu.get_barrier_semaphore()
        for t in peers:
            pl.semaphore_signal(
                barrier_sem,
                inc=1,
                device_id=(t,),
                device_id_type=pl.DeviceIdType.MESH,
            )
        pl.semaphore_wait(barrier_sem, len(peers))

        rdma_x = pltpu.make_async_remote_copy(
            src_ref=p_ref.at[pl.ds(8 * (1 - myx), 8)],
            dst_ref=rx_ref,
            send_sem=sx_send,
            recv_sem=sx_recv,
            device_id=(px,),
            device_id_type=pl.DeviceIdType.MESH,
        )
        rdma_x.start()
        rdma_x.wait_recv()
        mine = pl.ds(8 * myx, 8)
        p_ref[mine] = p_ref[mine] + rx_ref[...]

        rdma_y = pltpu.make_async_remote_copy(
            src_ref=p_ref.at[pl.ds(8 * myx + 4 * (1 - myy), 4)],
            dst_ref=ry_ref,
            send_sem=sy_send,
            recv_sem=sy_recv,
            device_id=(py,),
            device_id_type=pl.DeviceIdType.MESH,
        )
        rdma_y.start()
        rdma_y.wait_recv()
        col = pl.ds(8 * myx + 4 * myy, 4)
        p_ref[col] = p_ref[col] + ry_ref[...]

        base = 8 * myx + 4 * myy
        rdmas_z = []
        for j in range(3):
            zp = jnp.where(j >= myz, j + 1, j)
            slot = jnp.where(myz > zp, myz - 1, myz)
            rdma = pltpu.make_async_remote_copy(
                src_ref=p_ref.at[base + zp],
                dst_ref=rz_ref.at[slot],
                send_sem=sz_send.at[j],
                recv_sem=sz_recv.at[slot],
                device_id=(4 * zp + myq,),
                device_id_type=pl.DeviceIdType.MESH,
            )
            rdma.start()
            rdmas_z.append(rdma)

        acc = p_ref[base + myz].astype(jnp.float32)
        for k in range(3):
            recv = pltpu.make_async_remote_copy(
                src_ref=p_ref.at[0],
                dst_ref=rz_ref.at[k],
                send_sem=sz_send.at[0],
                recv_sem=sz_recv.at[k],
                device_id=(my,),
                device_id_type=pl.DeviceIdType.MESH,
            )
            recv.wait_recv()
            acc = acc + rz_ref[k].astype(jnp.float32)

        out_ref[...] = acc / (1.0 + jnp.exp(-acc))

        rdma_x.wait_send()
        rdma_y.wait_send()
        for rdma in rdmas_z:
            rdma.wait_send()

        @functools.partial(pl.run_scoped, exit_sem=pltpu.SemaphoreType.REGULAR)
        def _(exit_sem):
            for t in peers:
                pl.semaphore_signal(
                    exit_sem,
                    inc=1,
                    device_id=(t,),
                    device_id_type=pl.DeviceIdType.MESH,
                )
            pl.semaphore_wait(exit_sem, len(peers))

    return pl.pallas_call(
        body,
        out_shape=jax.ShapeDtypeStruct((rows, n), jnp.float32),
        in_specs=[
            pl.BlockSpec(memory_space=pltpu.VMEM),
            pl.BlockSpec(memory_space=pltpu.VMEM),
        ],
        out_specs=pl.BlockSpec(memory_space=pltpu.VMEM),
        scratch_shapes=[
            pltpu.VMEM((N_DEV, rows, n), jnp.bfloat16),
            pltpu.VMEM((8, rows, n), jnp.bfloat16),
            pltpu.VMEM((4, rows, n), jnp.bfloat16),
            pltpu.VMEM((3, rows, n), jnp.bfloat16),
            pltpu.SemaphoreType.DMA,
            pltpu.SemaphoreType.DMA,
            pltpu.SemaphoreType.DMA,
            pltpu.SemaphoreType.DMA,
            pltpu.SemaphoreType.DMA((3,)),
            pltpu.SemaphoreType.DMA((3,)),
        ],
        compiler_params=pltpu.CompilerParams(collective_id=0),
    )(x, w_mat)


# device time: 25678 ns/iter; 1.4720x vs baseline; 1.4644x over previous
import jax
import jax.numpy as jnp
from jax import lax
from jax.experimental import pallas as pl
from jax.experimental.pallas import tpu as pltpu

N_DEV = 16

_QXY = ((0, 0), (1, 0), (1, 1), (0, 1))
_XY2Q = {(x, y): q for q, (x, y) in enumerate(_QXY)}


def kernel(x, w_mat):
    m, _ = x.shape
    _, n = w_mat.shape
    rows = m // N_DEV
    nh = n // 2

    def body(
        x_ref,
        w_ref,
        out_ref,
        p_ref,
        rxa_ref,
        ryb_ref,
        rya_ref,
        rxb_ref,
        rz_ref,
        sax_s,
        sax_r,
        sby_s,
        sby_r,
        say_s,
        say_r,
        sbx_s,
        sbx_r,
        sz_send,
        sz_recv,
    ):
        my = lax.axis_index("i")
        myq = my % 4
        myz = my // 4
        myx = ((myq == 1) | (myq == 2)).astype(jnp.int32)
        myy = (myq >= 2).astype(jnp.int32)
        px = 4 * myz + (myq ^ 1)
        py = 4 * myz + (3 - myq)
        A = pl.ds(0, nh)
        B = pl.ds(nh, nh)

        p = jnp.dot(
            x_ref[...].astype(jnp.bfloat16),
            w_ref[...].astype(jnp.bfloat16),
            preferred_element_type=jnp.float32,
        ).astype(jnp.bfloat16)
        for ell in range(N_DEV):
            xg, yg, z = ell // 8, (ell % 8) // 4, ell % 4
            d = 4 * z + _XY2Q[(xg, yg)]
            p_ref[ell] = p[d * rows : (d + 1) * rows, :]

        zmates = [4 * jnp.where(j >= myz, j + 1, j) + myq for j in range(3)]
        peers = [px, py] + zmates
        barrier_sem = pltpu.get_barrier_semaphore()
        for t in peers:
            pl.semaphore_signal(
                barrier_sem,
                inc=1,
                device_id=(t,),
                device_id_type=pl.DeviceIdType.MESH,
            )
        pl.semaphore_wait(barrier_sem, len(peers))

        rdma_ax = pltpu.make_async_remote_copy(
            src_ref=p_ref.at[pl.ds(8 * (1 - myx), 8), :, A],
            dst_ref=rxa_ref,
            send_sem=sax_s,
            recv_sem=sax_r,
            device_id=(px,),
            device_id_type=pl.DeviceIdType.MESH,
        )
        rdma_ax.start()
        rdmas_by = []
        for g in range(2):
            rdma = pltpu.make_async_remote_copy(
                src_ref=p_ref.at[pl.ds(8 * g + 4 * (1 - myy), 4), :, B],
                dst_ref=ryb_ref.at[pl.ds(4 * g, 4)],
                send_sem=sby_s.at[g],
                recv_sem=sby_r.at[g],
                device_id=(py,),
                device_id_type=pl.DeviceIdType.MESH,
            )
            rdma.start()
            rdmas_by.append(rdma)

        rdma_ax.wait_recv()
        mine = pl.ds(8 * myx, 8)
        p_ref[mine, :, A] = p_ref[mine, :, A] + rxa_ref[...]
        rdma_ay = pltpu.make_async_remote_copy(
            src_ref=p_ref.at[pl.ds(8 * myx + 4 * (1 - myy), 4), :, A],
            dst_ref=rya_ref,
            send_sem=say_s,
            recv_sem=say_r,
            device_id=(py,),
            device_id_type=pl.DeviceIdType.MESH,
        )
        rdma_ay.start()

        for g in range(2):
            rdmas_by[g].wait_recv()
            sl = pl.ds(8 * g + 4 * myy, 4)
            p_ref[sl, :, B] = p_ref[sl, :, B] + ryb_ref[pl.ds(4 * g, 4)]
        rdma_bx = pltpu.make_async_remote_copy(
            src_ref=p_ref.at[pl.ds(8 * (1 - myx) + 4 * myy, 4), :, B],
            dst_ref=rxb_ref,
            send_sem=sbx_s,
            recv_sem=sbx_r,
            device_id=(px,),
            device_id_type=pl.DeviceIdType.MESH,
        )
        rdma_bx.start()

        col = pl.ds(8 * myx + 4 * myy, 4)
        rdma_ay.wait_recv()
        p_ref[col, :, A] = p_ref[col, :, A] + rya_ref[...]
        rdma_bx.wait_recv()
        p_ref[col, :, B] = p_ref[col, :, B] + rxb_ref[...]

        base = 8 * myx + 4 * myy
        rdmas_z = []
        for j in range(3):
            zp = jnp.where(j >= myz, j + 1, j)
            slot = jnp.where(myz > zp, myz - 1, myz)
            rdma = pltpu.make_async_remote_copy(
                src_ref=p_ref.at[base + zp],
                dst_ref=rz_ref.at[slot],
                send_sem=sz_send.at[j],
                recv_sem=sz_recv.at[slot],
                device_id=(4 * zp + myq,),
                device_id_type=pl.DeviceIdType.MESH,
            )
            rdma.start()
            rdmas_z.append(rdma)

        acc = p_ref[base + myz].astype(jnp.float32)
        for k in range(3):
            recv = pltpu.make_async_remote_copy(
                src_ref=p_ref.at[0],
                dst_ref=rz_ref.at[k],
                send_sem=sz_send.at[0],
                recv_sem=sz_recv.at[k],
                device_id=(my,),
                device_id_type=pl.DeviceIdType.MESH,
            )
            recv.wait_recv()
            acc = acc + rz_ref[k].astype(jnp.float32)

        out_ref[...] = acc / (1.0 + jnp.exp(-acc))

        rdma_ax.wait_send()
        for rdma in rdmas_by:
            rdma.wait_send()
        rdma_ay.wait_send()
        rdma_bx.wait_send()
        for rdma in rdmas_z:
            rdma.wait_send()

    return pl.pallas_call(
        body,
        out_shape=jax.ShapeDtypeStruct((rows, n), jnp.float32),
        in_specs=[
            pl.BlockSpec(memory_space=pltpu.VMEM),
            pl.BlockSpec(memory_space=pltpu.VMEM),
        ],
        out_specs=pl.BlockSpec(memory_space=pltpu.VMEM),
        scratch_shapes=[
            pltpu.VMEM((N_DEV, rows, n), jnp.bfloat16),
            pltpu.VMEM((8, rows, nh), jnp.bfloat16),
            pltpu.VMEM((8, rows, nh), jnp.bfloat16),
            pltpu.VMEM((4, rows, nh), jnp.bfloat16),
            pltpu.VMEM((4, rows, nh), jnp.bfloat16),
            pltpu.VMEM((3, rows, n), jnp.bfloat16),
            pltpu.SemaphoreType.DMA,
            pltpu.SemaphoreType.DMA,
            pltpu.SemaphoreType.DMA((2,)),
            pltpu.SemaphoreType.DMA((2,)),
            pltpu.SemaphoreType.DMA,
            pltpu.SemaphoreType.DMA,
            pltpu.SemaphoreType.DMA,
            pltpu.SemaphoreType.DMA,
            pltpu.SemaphoreType.DMA((3,)),
            pltpu.SemaphoreType.DMA((3,)),
        ],
        compiler_params=pltpu.CompilerParams(collective_id=0),
    )(x, w_mat)


# device time: 24408 ns/iter; 1.5486x vs baseline; 1.0520x over previous
import jax
import jax.numpy as jnp
from jax import lax
from jax.experimental import pallas as pl
from jax.experimental.pallas import tpu as pltpu

N_DEV = 16

_QXY = ((0, 0), (1, 0), (1, 1), (0, 1))
_XY2Q = {(x, y): q for q, (x, y) in enumerate(_QXY)}


def kernel(x, w_mat):
    m, _ = x.shape
    _, n = w_mat.shape
    rows = m // N_DEV
    nh = n // 2

    def body(
        x_ref,
        w_ref,
        out_ref,
        p_ref,
        rxa_ref,
        ryb_ref,
        rya_ref,
        rxb_ref,
        rz_ref,
        sax_s,
        sax_r,
        sby_s,
        sby_r,
        say_s,
        say_r,
        sbx_s,
        sbx_r,
        sz_send,
        sz_recv,
        zr_sem,
    ):
        my = lax.axis_index("i")
        myq = my % 4
        myz = my // 4
        myx = ((myq == 1) | (myq == 2)).astype(jnp.int32)
        myy = (myq >= 2).astype(jnp.int32)
        px = 4 * myz + (myq ^ 1)
        py = 4 * myz + (3 - myq)
        A = pl.ds(0, nh)
        B = pl.ds(nh, nh)

        p = jnp.dot(
            x_ref[...].astype(jnp.bfloat16),
            w_ref[...].astype(jnp.bfloat16),
            preferred_element_type=jnp.float32,
        ).astype(jnp.bfloat16)
        for ell in range(N_DEV):
            xg, yg, z = ell // 8, (ell % 8) // 4, ell % 4
            d = 4 * z + _XY2Q[(xg, yg)]
            p_ref[ell] = p[d * rows : (d + 1) * rows, :]

        zmates = [4 * jnp.where(j >= myz, j + 1, j) + myq for j in range(3)]
        barrier_sem = pltpu.get_barrier_semaphore()
        for t in [px, py]:
            pl.semaphore_signal(
                barrier_sem,
                inc=1,
                device_id=(t,),
                device_id_type=pl.DeviceIdType.MESH,
            )
        pl.semaphore_wait(barrier_sem, 2)
        for t in zmates:
            pl.semaphore_signal(
                zr_sem,
                inc=1,
                device_id=(t,),
                device_id_type=pl.DeviceIdType.MESH,
            )

        rdma_ax = pltpu.make_async_remote_copy(
            src_ref=p_ref.at[pl.ds(8 * (1 - myx), 8), :, A],
            dst_ref=rxa_ref,
            send_sem=sax_s,
            recv_sem=sax_r,
            device_id=(px,),
            device_id_type=pl.DeviceIdType.MESH,
        )
        rdma_ax.start()
        rdmas_by = []
        for g in range(2):
            rdma = pltpu.make_async_remote_copy(
                src_ref=p_ref.at[pl.ds(8 * g + 4 * (1 - myy), 4), :, B],
                dst_ref=ryb_ref.at[pl.ds(4 * g, 4)],
                send_sem=sby_s.at[g],
                recv_sem=sby_r.at[g],
                device_id=(py,),
                device_id_type=pl.DeviceIdType.MESH,
            )
            rdma.start()
            rdmas_by.append(rdma)

        rdma_ax.wait_recv()
        mine = pl.ds(8 * myx, 8)
        p_ref[mine, :, A] = p_ref[mine, :, A] + rxa_ref[...]
        rdma_ay = pltpu.make_async_remote_copy(
            src_ref=p_ref.at[pl.ds(8 * myx + 4 * (1 - myy), 4), :, A],
            dst_ref=rya_ref,
            send_sem=say_s,
            recv_sem=say_r,
            device_id=(py,),
            device_id_type=pl.DeviceIdType.MESH,
        )
        rdma_ay.start()

        for g in range(2):
            rdmas_by[g].wait_recv()
            sl = pl.ds(8 * g + 4 * myy, 4)
            p_ref[sl, :, B] = p_ref[sl, :, B] + ryb_ref[pl.ds(4 * g, 4)]
        rdma_bx = pltpu.make_async_remote_copy(
            src_ref=p_ref.at[pl.ds(8 * (1 - myx) + 4 * myy, 4), :, B],
            dst_ref=rxb_ref,
            send_sem=sbx_s,
            recv_sem=sbx_r,
            device_id=(px,),
            device_id_type=pl.DeviceIdType.MESH,
        )
        rdma_bx.start()

        col = pl.ds(8 * myx + 4 * myy, 4)
        rdma_ay.wait_recv()
        p_ref[col, :, A] = p_ref[col, :, A] + rya_ref[...]
        rdma_bx.wait_recv()
        p_ref[col, :, B] = p_ref[col, :, B] + rxb_ref[...]

        base = 8 * myx + 4 * myy
        pl.semaphore_wait(zr_sem, 3)
        rdmas_z = []
        for j in range(3):
            zp = jnp.where(j >= myz, j + 1, j)
            slot = jnp.where(myz > zp, myz - 1, myz)
            rdma = pltpu.make_async_remote_copy(
                src_ref=p_ref.at[base + zp],
                dst_ref=rz_ref.at[slot],
                send_sem=sz_send.at[j],
                recv_sem=sz_recv.at[slot],
                device_id=(4 * zp + myq,),
                device_id_type=pl.DeviceIdType.MESH,
            )
            rdma.start()
            rdmas_z.append(rdma)

        acc = p_ref[base + myz].astype(jnp.float32)
        for k in range(3):
            recv = pltpu.make_async_remote_copy(
                src_ref=p_ref.at[0],
                dst_ref=rz_ref.at[k],
                send_sem=sz_send.at[0],
                recv_sem=sz_recv.at[k],
                device_id=(my,),
                device_id_type=pl.DeviceIdType.MESH,
            )
            recv.wait_recv()
            acc = acc + rz_ref[k].astype(jnp.float32)

        out_ref[...] = acc / (1.0 + jnp.exp(-acc))

        rdma_ax.wait_send()
        for rdma in rdmas_by:
            rdma.wait_send()
        rdma_ay.wait_send()
        rdma_bx.wait_send()
        for rdma in rdmas_z:
            rdma.wait_send()

    return pl.pallas_call(
        body,
        out_shape=jax.ShapeDtypeStruct((rows, n), jnp.float32),
        in_specs=[
            pl.BlockSpec(memory_space=pltpu.VMEM),
            pl.BlockSpec(memory_space=pltpu.VMEM),
        ],
        out_specs=pl.BlockSpec(memory_space=pltpu.VMEM),
        scratch_shapes=[
            pltpu.VMEM((N_DEV, rows, n), jnp.bfloat16),
            pltpu.VMEM((8, rows, nh), jnp.bfloat16),
            pltpu.VMEM((8, rows, nh), jnp.bfloat16),
            pltpu.VMEM((4, rows, nh), jnp.bfloat16),
            pltpu.VMEM((4, rows, nh), jnp.bfloat16),
            pltpu.VMEM((3, rows, n), jnp.bfloat16),
            pltpu.SemaphoreType.DMA,
            pltpu.SemaphoreType.DMA,
            pltpu.SemaphoreType.DMA((2,)),
            pltpu.SemaphoreType.DMA((2,)),
            pltpu.SemaphoreType.DMA,
            pltpu.SemaphoreType.DMA,
            pltpu.SemaphoreType.DMA,
            pltpu.SemaphoreType.DMA,
            pltpu.SemaphoreType.DMA((3,)),
            pltpu.SemaphoreType.DMA((3,)),
            pltpu.SemaphoreType.REGULAR,
        ],
        compiler_params=pltpu.CompilerParams(collective_id=0),
    )(x, w_mat)


# device time: 23419 ns/iter; 1.6140x vs baseline; 1.0422x over previous
import jax
import jax.numpy as jnp
from jax import lax
from jax.experimental import pallas as pl
from jax.experimental.pallas import tpu as pltpu

N_DEV = 16

_QXY = ((0, 0), (1, 0), (1, 1), (0, 1))
_XY2Q = {(x, y): q for q, (x, y) in enumerate(_QXY)}


def kernel(x, w_mat):
    m, _ = x.shape
    _, n = w_mat.shape
    rows = m // N_DEV
    nh = n // 2

    def body(
        x_ref,
        w_ref,
        out_ref,
        p_ref,
        rxa_ref,
        ryb_ref,
        rya_ref,
        rxb_ref,
        rz_ref,
        sax_s,
        sax_r,
        sby_s,
        sby_r,
        say_s,
        say_r,
        sbx_s,
        sbx_r,
        sz_send,
        sz_recv,
        zr_sem,
    ):
        my = lax.axis_index("i")
        myq = my % 4
        myz = my // 4
        myx = ((myq == 1) | (myq == 2)).astype(jnp.int32)
        myy = (myq >= 2).astype(jnp.int32)
        px = 4 * myz + (myq ^ 1)
        py = 4 * myz + (3 - myq)
        A = pl.ds(0, nh)
        B = pl.ds(nh, nh)

        p = jnp.dot(
            x_ref[...].astype(jnp.bfloat16),
            w_ref[...].astype(jnp.bfloat16),
            preferred_element_type=jnp.float32,
        ).astype(jnp.bfloat16)
        for ell in range(N_DEV):
            xg, yg, z = ell // 8, (ell % 8) // 4, ell % 4
            d = 4 * z + _XY2Q[(xg, yg)]
            p_ref[ell] = p[d * rows : (d + 1) * rows, :]

        zmates = [4 * jnp.where(j >= myz, j + 1, j) + myq for j in range(3)]
        barrier_sem = pltpu.get_barrier_semaphore()
        for t in [px, py]:
            pl.semaphore_signal(
                barrier_sem,
                inc=1,
                device_id=(t,),
                device_id_type=pl.DeviceIdType.MESH,
            )
        pl.semaphore_wait(barrier_sem, 2)
        for t in zmates:
            pl.semaphore_signal(
                zr_sem,
                inc=1,
                device_id=(t,),
                device_id_type=pl.DeviceIdType.MESH,
            )

        rdmas_ax = []
        for j in range(2):
            off = (1 - myy) if j == 0 else myy
            rdma = pltpu.make_async_remote_copy(
                src_ref=p_ref.at[pl.ds(8 * (1 - myx) + 4 * off, 4), :, A],
                dst_ref=rxa_ref.at[pl.ds(4 * off, 4)],
                send_sem=sax_s.at[j],
                recv_sem=sax_r.at[j],
                device_id=(px,),
                device_id_type=pl.DeviceIdType.MESH,
            )
            rdma.start()
            rdmas_ax.append(rdma)
        rdmas_by = []
        for j in range(2):
            g = (1 - myx) ^ j
            rdma = pltpu.make_async_remote_copy(
                src_ref=p_ref.at[pl.ds(8 * g + 4 * (1 - myy), 4), :, B],
                dst_ref=ryb_ref.at[pl.ds(4 * g, 4)],
                send_sem=sby_s.at[j],
                recv_sem=sby_r.at[j],
                device_id=(py,),
                device_id_type=pl.DeviceIdType.MESH,
            )
            rdma.start()
            rdmas_by.append(rdma)

        rdmas_ax[0].wait_recv()
        yo = pl.ds(8 * myx + 4 * (1 - myy), 4)
        p_ref[yo, :, A] = p_ref[yo, :, A] + rxa_ref[pl.ds(4 * (1 - myy), 4)]
        rdma_ay = pltpu.make_async_remote_copy(
            src_ref=p_ref.at[yo, :, A],
            dst_ref=rya_ref,
            send_sem=say_s,
            recv_sem=say_r,
            device_id=(py,),
            device_id_type=pl.DeviceIdType.MESH,
        )
        rdma_ay.start()

        rdmas_by[0].wait_recv()
        xo = pl.ds(8 * (1 - myx) + 4 * myy, 4)
        p_ref[xo, :, B] = p_ref[xo, :, B] + ryb_ref[pl.ds(4 * (1 - myx), 4)]
        rdma_bx = pltpu.make_async_remote_copy(
            src_ref=p_ref.at[xo, :, B],
            dst_ref=rxb_ref,
            send_sem=sbx_s,
            recv_sem=sbx_r,
            device_id=(px,),
            device_id_type=pl.DeviceIdType.MESH,
        )
        rdma_bx.start()

        col = pl.ds(8 * myx + 4 * myy, 4)
        rdmas_ax[1].wait_recv()
        p_ref[col, :, A] = p_ref[col, :, A] + rxa_ref[pl.ds(4 * myy, 4)]
        rdmas_by[1].wait_recv()
        p_ref[col, :, B] = p_ref[col, :, B] + ryb_ref[pl.ds(4 * myx, 4)]

        rdma_ay.wait_recv()
        p_ref[col, :, A] = p_ref[col, :, A] + rya_ref[...]
        rdma_bx.wait_recv()
        p_ref[col, :, B] = p_ref[col, :, B] + rxb_ref[...]

        base = 8 * myx + 4 * myy
        pl.semaphore_wait(zr_sem, 3)
        rdmas_z = []
        for j in range(3):
            zp = jnp.where(j >= myz, j + 1, j)
            slot = jnp.where(myz > zp, myz - 1, myz)
            rdma = pltpu.make_async_remote_copy(
                src_ref=p_ref.at[base + zp],
                dst_ref=rz_ref.at[slot],
                send_sem=sz_send.at[j],
                recv_sem=sz_recv.at[slot],
                device_id=(4 * zp + myq,),
                device_id_type=pl.DeviceIdType.MESH,
            )
            rdma.start()
            rdmas_z.append(rdma)

        acc = p_ref[base + myz].astype(jnp.float32)
        for k in range(3):
            recv = pltpu.make_async_remote_copy(
                src_ref=p_ref.at[0],
                dst_ref=rz_ref.at[k],
                send_sem=sz_send.at[0],
                recv_sem=sz_recv.at[k],
                device_id=(my,),
                device_id_type=pl.DeviceIdType.MESH,
            )
            recv.wait_recv()
            acc = acc + rz_ref[k].astype(jnp.float32)

        out_ref[...] = acc / (1.0 + jnp.exp(-acc))

        for rdma in rdmas_ax:
            rdma.wait_send()
        for rdma in rdmas_by:
            rdma.wait_send()
        rdma_ay.wait_send()
        rdma_bx.wait_send()
        for rdma in rdmas_z:
            rdma.wait_send()

    return pl.pallas_call(
        body,
        out_shape=jax.ShapeDtypeStruct((rows, n), jnp.float32),
        in_specs=[
            pl.BlockSpec(memory_space=pltpu.VMEM),
            pl.BlockSpec(memory_space=pltpu.VMEM),
        ],
        out_specs=pl.BlockSpec(memory_space=pltpu.VMEM),
        scratch_shapes=[
            pltpu.VMEM((N_DEV, rows, n), jnp.bfloat16),
            pltpu.VMEM((8, rows, nh), jnp.bfloat16),
            pltpu.VMEM((8, rows, nh), jnp.bfloat16),
            pltpu.VMEM((4, rows, nh), jnp.bfloat16),
            pltpu.VMEM((4, rows, nh), jnp.bfloat16),
            pltpu.VMEM((3, rows, n), jnp.bfloat16),
            pltpu.SemaphoreType.DMA((2,)),
            pltpu.SemaphoreType.DMA((2,)),
            pltpu.SemaphoreType.DMA((2,)),
            pltpu.SemaphoreType.DMA((2,)),
            pltpu.SemaphoreType.DMA,
            pltpu.SemaphoreType.DMA,
            pltpu.SemaphoreType.DMA,
            pltpu.SemaphoreType.DMA,
            pltpu.SemaphoreType.DMA((3,)),
            pltpu.SemaphoreType.DMA((3,)),
            pltpu.SemaphoreType.REGULAR,
        ],
        compiler_params=pltpu.CompilerParams(collective_id=0),
    )(x, w_mat)


# device time: 22673 ns/iter; 1.6671x vs baseline; 1.0329x over previous
import jax
import jax.numpy as jnp
from jax import lax
from jax.experimental import pallas as pl
from jax.experimental.pallas import tpu as pltpu

N_DEV = 16

_QXY = ((0, 0), (1, 0), (1, 1), (0, 1))
_XY2Q = {(x, y): q for q, (x, y) in enumerate(_QXY)}


def kernel(x, w_mat):
    m, _ = x.shape
    _, n = w_mat.shape
    rows = m // N_DEV
    nh = n // 2

    def body(
        x_ref,
        w_ref,
        out_ref,
        p_ref,
        xp_ref,
        rxa_ref,
        ryb_ref,
        rya_ref,
        rxb_ref,
        rz_ref,
        sax_s,
        sax_r,
        sby_s,
        sby_r,
        say_s,
        say_r,
        sbx_s,
        sbx_r,
        sz_send,
        sz_recv,
        zr_sem,
    ):
        my = lax.axis_index("i")
        myq = my % 4
        myz = my // 4
        myx = ((myq == 1) | (myq == 2)).astype(jnp.int32)
        myy = (myq >= 2).astype(jnp.int32)
        px = 4 * myz + (myq ^ 1)
        py = 4 * myz + (3 - myq)
        A = pl.ds(0, nh)
        B = pl.ds(nh, nh)

        zmates = [4 * jnp.where(j >= myz, j + 1, j) + myq for j in range(3)]
        barrier_sem = pltpu.get_barrier_semaphore()
        for t in [px, py]:
            pl.semaphore_signal(
                barrier_sem,
                inc=1,
                device_id=(t,),
                device_id_type=pl.DeviceIdType.MESH,
            )
        for t in zmates:
            pl.semaphore_signal(
                zr_sem,
                inc=1,
                device_id=(t,),
                device_id_type=pl.DeviceIdType.MESH,
            )

        for ell in range(N_DEV):
            xg, yg, z = ell // 8, (ell % 8) // 4, ell % 4
            d = 4 * z + _XY2Q[(xg, yg)]
            xp_ref[ell * rows : (ell + 1) * rows, :] = x_ref[
                d * rows : (d + 1) * rows, :
            ].astype(jnp.bfloat16)
        wb = w_ref[...].astype(jnp.bfloat16)

        oth_rows = pl.ds(512 * (1 - myx), 512)
        p_ref[pl.ds(8 * (1 - myx), 8)] = (
            jnp.dot(xp_ref[oth_rows, :], wb, preferred_element_type=jnp.float32)
            .astype(jnp.bfloat16)
            .reshape(8, rows, n)
        )

        pl.semaphore_wait(barrier_sem, 2)

        rdmas_ax = []
        for j in range(2):
            off = (1 - myy) if j == 0 else myy
            rdma = pltpu.make_async_remote_copy(
                src_ref=p_ref.at[pl.ds(8 * (1 - myx) + 4 * off, 4), :, A],
                dst_ref=rxa_ref.at[pl.ds(4 * off, 4)],
                send_sem=sax_s.at[j],
                recv_sem=sax_r.at[j],
                device_id=(px,),
                device_id_type=pl.DeviceIdType.MESH,
            )
            rdma.start()
            rdmas_ax.append(rdma)
        rdmas_by = []
        for j in range(2):
            g = (1 - myx) ^ j
            rdma = pltpu.make_async_remote_copy(
                src_ref=p_ref.at[pl.ds(8 * g + 4 * (1 - myy), 4), :, B],
                dst_ref=ryb_ref.at[pl.ds(4 * g, 4)],
                send_sem=sby_s.at[j],
                recv_sem=sby_r.at[j],
                device_id=(py,),
                device_id_type=pl.DeviceIdType.MESH,
            )
            rdmas_by.append(rdma)
        rdmas_by[0].start()

        mine_rows = pl.ds(512 * myx, 512)
        p_ref[pl.ds(8 * myx, 8)] = (
            jnp.dot(xp_ref[mine_rows, :], wb, preferred_element_type=jnp.float32)
            .astype(jnp.bfloat16)
            .reshape(8, rows, n)
        )
        rdmas_by[1].start()

        rdmas_ax[0].wait_recv()
        yo = pl.ds(8 * myx + 4 * (1 - myy), 4)
        p_ref[yo, :, A] = p_ref[yo, :, A] + rxa_ref[pl.ds(4 * (1 - myy), 4)]
        rdma_ay = pltpu.make_async_remote_copy(
            src_ref=p_ref.at[yo, :, A],
            dst_ref=rya_ref,
            send_sem=say_s,
            recv_sem=say_r,
            device_id=(py,),
            device_id_type=pl.DeviceIdType.MESH,
        )
        rdma_ay.start()

        rdmas_by[0].wait_recv()
        xo = pl.ds(8 * (1 - myx) + 4 * myy, 4)
        p_ref[xo, :, B] = p_ref[xo, :, B] + ryb_ref[pl.ds(4 * (1 - myx), 4)]
        rdma_bx = pltpu.make_async_remote_copy(
            src_ref=p_ref.at[xo, :, B],
            dst_ref=rxb_ref,
            send_sem=sbx_s,
            recv_sem=sbx_r,
            device_id=(px,),
            device_id_type=pl.DeviceIdType.MESH,
        )
        rdma_bx.start()

        col = pl.ds(8 * myx + 4 * myy, 4)
        rdmas_ax[1].wait_recv()
        p_ref[col, :, A] = p_ref[col, :, A] + rxa_ref[pl.ds(4 * myy, 4)]
        rdmas_by[1].wait_recv()
        p_ref[col, :, B] = p_ref[col, :, B] + ryb_ref[pl.ds(4 * myx, 4)]

        rdma_ay.wait_recv()
        p_ref[col, :, A] = p_ref[col, :, A] + rya_ref[...]
        rdma_bx.wait_recv()
        p_ref[col, :, B] = p_ref[col, :, B] + rxb_ref[...]

        base = 8 * myx + 4 * myy
        pl.semaphore_wait(zr_sem, 3)
        rdmas_z = []
        for j in range(3):
            zp = jnp.where(j >= myz, j + 1, j)
            slot = jnp.where(myz > zp, myz - 1, myz)
            rdma = pltpu.make_async_remote_copy(
                src_ref=p_ref.at[base + zp],
                dst_ref=rz_ref.at[slot],
                send_sem=sz_send.at[j],
                recv_sem=sz_recv.at[slot],
                device_id=(4 * zp + myq,),
                device_id_type=pl.DeviceIdType.MESH,
            )
            rdma.start()
            rdmas_z.append(rdma)

        acc = p_ref[base + myz].astype(jnp.float32)
        for k in range(3):
            recv = pltpu.make_async_remote_copy(
                src_ref=p_ref.at[0],
                dst_ref=rz_ref.at[k],
                send_sem=sz_send.at[0],
                recv_sem=sz_recv.at[k],
                device_id=(my,),
                device_id_type=pl.DeviceIdType.MESH,
            )
            recv.wait_recv()
            acc = acc + rz_ref[k].astype(jnp.float32)

        out_ref[...] = acc / (1.0 + jnp.exp(-acc))

        for rdma in rdmas_ax:
            rdma.wait_send()
        for rdma in rdmas_by:
            rdma.wait_send()
        rdma_ay.wait_send()
        rdma_bx.wait_send()
        for rdma in rdmas_z:
            rdma.wait_send()

    return pl.pallas_call(
        body,
        out_shape=jax.ShapeDtypeStruct((rows, n), jnp.float32),
        in_specs=[
            pl.BlockSpec(memory_space=pltpu.VMEM),
            pl.BlockSpec(memory_space=pltpu.VMEM),
        ],
        out_specs=pl.BlockSpec(memory_space=pltpu.VMEM),
        scratch_shapes=[
            pltpu.VMEM((N_DEV, rows, n), jnp.bfloat16),
            pltpu.VMEM((m, x.shape[1]), jnp.bfloat16),
            pltpu.VMEM((8, rows, nh), jnp.bfloat16),
            pltpu.VMEM((8, rows, nh), jnp.bfloat16),
            pltpu.VMEM((4, rows, nh), jnp.bfloat16),
            pltpu.VMEM((4, rows, nh), jnp.bfloat16),
            pltpu.VMEM((3, rows, n), jnp.bfloat16),
            pltpu.SemaphoreType.DMA((2,)),
            pltpu.SemaphoreType.DMA((2,)),
            pltpu.SemaphoreType.DMA((2,)),
            pltpu.SemaphoreType.DMA((2,)),
            pltpu.SemaphoreType.DMA,
            pltpu.SemaphoreType.DMA,
            pltpu.SemaphoreType.DMA,
            pltpu.SemaphoreType.DMA,
            pltpu.SemaphoreType.DMA((3,)),
            pltpu.SemaphoreType.DMA((3,)),
            pltpu.SemaphoreType.REGULAR,
        ],
        compiler_params=pltpu.CompilerParams(collective_id=0),
    )(x, w_mat)
